# baseline (speedup 1.0000x reference)
import functools

import jax
import jax.numpy as jnp
from jax import lax
from jax.experimental import pallas as pl
from jax.experimental.pallas import tpu as pltpu
from jax.experimental.pallas import tpu_sc as plsc

B = 16
P = 400
Q = 30
V = 100000
D = 128
H = 256

NP_TOK = B * P
NQ_TOK = B * Q
Q_PAD = 512
NTOK = NP_TOK + Q_PAD

CH = 80
NCHP = P // CH

_INTERPRET = False

try:
    _info = plsc.get_sparse_core_info()
    _NUM_WORKERS = _info.num_cores * _info.num_subcores
except Exception:
    _NUM_WORKERS = 32


def _gather_rows(table, idx):
    n_workers = _NUM_WORKERS
    per_w = NTOK // n_workers
    chunk = 72
    n_ch = per_w // chunk

    mesh = plsc.VectorSubcoreMesh(core_axis_name="c", subcore_axis_name="s")

    @functools.partial(
        pl.kernel,
        mesh=mesh,
        out_type=jax.ShapeDtypeStruct((NTOK, D), jnp.float32),
        scratch_types=[
            pltpu.VMEM((n_ch, chunk), jnp.int32),
            pltpu.VMEM((per_w, D), jnp.float32),
            pltpu.SemaphoreType.DMA,
        ],
    )
    def gk(table_hbm, idx_hbm, out_hbm, idx_v, rows_v, sem):
        n_cores = n_workers // 16
        wid = lax.axis_index("s") * n_cores + lax.axis_index("c")
        base = wid * per_w
        for j in range(n_ch):
            pltpu.sync_copy(idx_hbm.at[pl.ds(base + j * chunk, chunk)], idx_v.at[j])
        copies = []
        for j in range(n_ch):
            copies.append(
                pltpu.async_copy(
                    table_hbm.at[idx_v.at[j]],
                    rows_v.at[pl.ds(j * chunk, chunk)],
                    sem,
                )
            )
        for c in copies:
            c.wait()
        pltpu.sync_copy(rows_v, out_hbm.at[pl.ds(base, per_w)])

    return gk(table, idx)


def _dot(a, b):
    return lax.dot_general(
        a, b, (((1,), (0,)), ((), ())),
        preferred_element_type=jnp.float32,
    )


def _dot_t(a, b):
    return lax.dot_general(
        a, b, (((1,), (1,)), ((), ())),
        preferred_element_type=jnp.float32,
    )


def _lstm_step(x, h, c, wh, b):
    z = (x + _dot(h, wh)) + b
    i = jax.nn.sigmoid(z[:, 0:H])
    f = jax.nn.sigmoid(z[:, H:2 * H])
    g = jnp.tanh(z[:, 2 * H:3 * H])
    o = jax.nn.sigmoid(z[:, 3 * H:4 * H])
    c = f * c + i * g
    h = o * jnp.tanh(c)
    return h, c


def _fused(rows, rows_q,
           wxpf, wxpb, whpf, whpb, bpf, bpb,
           wxqf, wxqb, whqf, whqb, bqf, bqb,
           w_att, w_m, b_m2, w_se):
    def body(xf_ref, xb_ref, xfn_ref, xbn_ref, rq_ref,
             wxpf_ref, wxpb_ref, whpf_ref, whpb_ref, bpf_ref, bpb_ref,
             wxqf_ref, wxqb_ref, whqf_ref, whqb_ref, bqf_ref, bqb_ref,
             wa_ref, wm_ref, bm_ref, wse_ref,
             lo_ref, pr_ref,
             hpf_s, hpb_s, hqf_s, hqb_s, xwfa_s, xwba_s, xwfb_s, xwbb_s,
             hf_s, cf_s, hb_s, cb_s):
        i = pl.program_id(0)

        @pl.when(i == 0)
        def _init():
            z = jnp.zeros((B, H), jnp.float32)
            hf_s[...] = z
            cf_s[...] = z
            hb_s[...] = z
            cb_s[...] = z
            xwfa_s[...] = _dot(xf_ref[...], wxpf_ref[...])
            xwba_s[...] = _dot(xb_ref[...], wxpb_ref[...])

        def _scan_chunk(xwf_s, xwb_s, xwfn_s, xwbn_s):
            hf, cf, hb, cb = hf_s[...], cf_s[...], hb_s[...], cb_s[...]
            whf = whpf_ref[...]
            whb = whpb_ref[...]
            bf = bpf_ref[...]
            bb = bpb_ref[...]
            for tl in range(CH):
                hf, cf = _lstm_step(xwf_s[tl * B:(tl + 1) * B, :], hf, cf, whf, bf)
                hpf_s[i, :, tl, :] = hf
                tb = CH - 1 - tl
                hb, cb = _lstm_step(xwb_s[tb * B:(tb + 1) * B, :], hb, cb, whb, bb)
                hpb_s[NCHP - 1 - i, :, tb, :] = hb
            hf_s[...] = hf
            cf_s[...] = cf
            hb_s[...] = hb
            cb_s[...] = cb
            xwfn_s[...] = _dot(xfn_ref[...], wxpf_ref[...])
            xwbn_s[...] = _dot(xbn_ref[...], wxpb_ref[...])

        @pl.when(jnp.logical_and(i < NCHP, i % 2 == 0))
        def _scan_even():
            _scan_chunk(xwfa_s, xwba_s, xwfb_s, xwbb_s)

        @pl.when(jnp.logical_and(i < NCHP, i % 2 == 1))
        def _scan_odd():
            _scan_chunk(xwfb_s, xwbb_s, xwfa_s, xwba_s)

        @pl.when(i == NCHP)
        def _q_and_attention():
            rq = rq_ref[...].reshape(Q * B, D)
            xwfa_s[0:Q * B, :] = _dot(rq, wxqf_ref[...])
            xwba_s[0:Q * B, :] = _dot(rq, wxqb_ref[...])
            z = jnp.zeros((B, H), jnp.float32)
            hf, cf, hb, cb = z, z, z, z
            whf = whqf_ref[...]
            whb = whqb_ref[...]
            bf = bqf_ref[...]
            bb = bqb_ref[...]
            for tl in range(Q):
                hf, cf = _lstm_step(xwfa_s[tl * B:(tl + 1) * B, :], hf, cf, whf, bf)
                hqf_s[:, tl, :] = hf
                tb = Q - 1 - tl
                hb, cb = _lstm_step(xwba_s[tb * B:(tb + 1) * B, :], hb, cb, whb, bb)
                hqb_s[:, tb, :] = hb

            wa = wa_ref[...]
            wm = wm_ref[...]
            bm = bm_ref[...]
            wse = wse_ref[...]
            iota = lax.broadcasted_iota(jnp.int32, (2, P), 1)
            GB = 4
            for b0 in range(0, B, GB):
                cats = []
                for b in range(b0, b0 + GB):
                    hp = jnp.concatenate(
                        [hpf_s[:, b, :, :].reshape(P, H),
                         hpb_s[:, b, :, :].reshape(P, H)], axis=-1)
                    hq = jnp.concatenate([hqf_s[b], hqb_s[b]], axis=-1)
                    tmp = _dot_t(hq, wa)
                    scores = _dot_t(hp, tmp)
                    mx = jnp.max(scores, axis=-1, keepdims=True)
                    e = jnp.exp(scores - mx)
                    alpha = e / jnp.sum(e, axis=-1, keepdims=True)
                    ctx = _dot(alpha, hq)
                    cats.append(jnp.concatenate([hp, ctx], axis=-1))
                mg = jnp.tanh(_dot(jnp.concatenate(cats, axis=0), wm) + bm)
                ltg = lax.dot_general(
                    wse, mg, (((0,), (1,)), ((), ())),
                    preferred_element_type=jnp.float32,
                )
                for k in range(GB):
                    lt = ltg[:, k * P:(k + 1) * P]
                    lo_ref[b0 + k, :, :] = lt
                    mx2 = jnp.max(lt, axis=-1, keepdims=True)
                    idx = jnp.min(jnp.where(lt == mx2, iota, P), axis=-1)
                    pr_ref[b0 + k, :, :] = idx.reshape(1, 2)

    c0 = lambda i: (0, 0)
    c03 = lambda i: (0, 0, 0)
    return pl.pallas_call(
        body,
        grid=(NCHP + 1,),
        in_specs=[
            pl.BlockSpec((CH * B, D), lambda i: (jnp.minimum(i, NCHP - 1), 0)),
            pl.BlockSpec((CH * B, D),
                         lambda i: (NCHP - 1 - jnp.minimum(i, NCHP - 1), 0)),
            pl.BlockSpec((CH * B, D), lambda i: (jnp.minimum(i + 1, NCHP - 1), 0)),
            pl.BlockSpec((CH * B, D),
                         lambda i: (jnp.clip(NCHP - 2 - i, 0, NCHP - 1), 0)),
            pl.BlockSpec((Q, B, D), c03),
            pl.BlockSpec((D, 4 * H), c0),
            pl.BlockSpec((D, 4 * H), c0),
            pl.BlockSpec((H, 4 * H), c0),
            pl.BlockSpec((H, 4 * H), c0),
            pl.BlockSpec((1, 4 * H), c0),
            pl.BlockSpec((1, 4 * H), c0),
            pl.BlockSpec((D, 4 * H), c0),
            pl.BlockSpec((D, 4 * H), c0),
            pl.BlockSpec((H, 4 * H), c0),
            pl.BlockSpec((H, 4 * H), c0),
            pl.BlockSpec((1, 4 * H), c0),
            pl.BlockSpec((1, 4 * H), c0),
            pl.BlockSpec((2 * H, 2 * H), c0),
            pl.BlockSpec((4 * H, 2 * H), c0),
            pl.BlockSpec((1, 2 * H), c0),
            pl.BlockSpec((2 * H, 2), c0),
        ],
        out_specs=[
            pl.BlockSpec((B, 2, P), c03),
            pl.BlockSpec((B, 1, 2), c03),
        ],
        out_shape=[
            jax.ShapeDtypeStruct((B, 2, P), jnp.float32),
            jax.ShapeDtypeStruct((B, 1, 2), jnp.int32),
        ],
        scratch_shapes=[
            pltpu.VMEM((NCHP, B, CH, H), jnp.float32),
            pltpu.VMEM((NCHP, B, CH, H), jnp.float32),
            pltpu.VMEM((B, Q, H), jnp.float32),
            pltpu.VMEM((B, Q, H), jnp.float32),
            pltpu.VMEM((CH * B, 4 * H), jnp.float32),
            pltpu.VMEM((CH * B, 4 * H), jnp.float32),
            pltpu.VMEM((CH * B, 4 * H), jnp.float32),
            pltpu.VMEM((CH * B, 4 * H), jnp.float32),
            pltpu.VMEM((B, H), jnp.float32),
            pltpu.VMEM((B, H), jnp.float32),
            pltpu.VMEM((B, H), jnp.float32),
            pltpu.VMEM((B, H), jnp.float32),
        ],
        interpret=_INTERPRET,
    )(rows, rows, rows, rows, rows_q,
      wxpf, wxpb, whpf, whpb, bpf, bpb,
      wxqf, wxqb, whqf, whqb, bqf, bqb,
      w_att, w_m, b_m2, w_se)


def kernel(passage, question, embedding,
           p_Wx_f, p_Wh_f, p_b_f, p_Wx_b, p_Wh_b, p_b_b,
           q_Wx_f, q_Wh_f, q_b_f, q_Wx_b, q_Wh_b, q_b_b,
           W_att, W_m, b_m, w_start, w_end):
    pidx = jnp.transpose(passage).reshape(-1).astype(jnp.int32)
    qidx = jnp.transpose(question[:, ::-1]).reshape(-1).astype(jnp.int32)
    idx = jnp.concatenate([pidx, qidx, jnp.zeros((Q_PAD - NQ_TOK,), jnp.int32)])

    rows = _gather_rows(embedding, idx)
    rows_q = rows[NP_TOK:NP_TOK + NQ_TOK].reshape(Q, B, D)

    logits, preds = _fused(
        rows, rows_q,
        p_Wx_f, p_Wx_b, p_Wh_f, p_Wh_b,
        p_b_f.reshape(1, 4 * H), p_b_b.reshape(1, 4 * H),
        q_Wx_f, q_Wx_b, q_Wh_f, q_Wh_b,
        q_b_f.reshape(1, 4 * H), q_b_b.reshape(1, 4 * H),
        W_att, W_m, b_m.reshape(1, 2 * H), jnp.stack([w_start, w_end], axis=1),
    )
    return logits, preds.reshape(B, 2)


# speedup vs baseline: 2.3924x; 2.3924x over previous
import functools

import jax
import jax.numpy as jnp
from jax import lax
from jax.experimental import pallas as pl
from jax.experimental.pallas import tpu as pltpu
from jax.experimental.pallas import tpu_sc as plsc

B = 16
P = 400
Q = 30
V = 100000
D = 128
H = 256

NP_TOK = B * P
NQ_TOK = B * Q
Q_PAD = 512
NTOK = NP_TOK + Q_PAD

CH = 80
NCHP = P // CH

_INTERPRET = False

try:
    _info = plsc.get_sparse_core_info()
    _NUM_WORKERS = _info.num_cores * _info.num_subcores
except Exception:
    _NUM_WORKERS = 32


def _gather_rows(table, idx):
    n_workers = _NUM_WORKERS
    per_w = NTOK // n_workers
    chunk = 72
    n_ch = per_w // chunk

    mesh = plsc.VectorSubcoreMesh(core_axis_name="c", subcore_axis_name="s")

    @functools.partial(
        pl.kernel,
        mesh=mesh,
        out_type=jax.ShapeDtypeStruct((NTOK, D), jnp.float32),
        scratch_types=[
            pltpu.VMEM((n_ch, chunk), jnp.int32),
            pltpu.VMEM((per_w, D), jnp.float32),
            pltpu.SemaphoreType.DMA,
        ],
    )
    def gk(table_hbm, idx_hbm, out_hbm, idx_v, rows_v, sem):
        n_cores = n_workers // 16
        wid = lax.axis_index("s") * n_cores + lax.axis_index("c")
        base = wid * per_w
        for j in range(n_ch):
            pltpu.sync_copy(idx_hbm.at[pl.ds(base + j * chunk, chunk)], idx_v.at[j])
        copies = []
        for j in range(n_ch):
            copies.append(
                pltpu.async_copy(
                    table_hbm.at[idx_v.at[j]],
                    rows_v.at[pl.ds(j * chunk, chunk)],
                    sem,
                )
            )
        for c in copies:
            c.wait()
        pltpu.sync_copy(rows_v, out_hbm.at[pl.ds(base, per_w)])

    return gk(table, idx)


def _dot(a, b):
    return lax.dot_general(
        a, b, (((1,), (0,)), ((), ())),
        preferred_element_type=jnp.float32,
    )


def _dot_t(a, b):
    return lax.dot_general(
        a, b, (((1,), (1,)), ((), ())),
        preferred_element_type=jnp.float32,
    )


def _lstm_step(x, h, c, wh, b):
    z = (x + _dot(h, wh)) + b
    i = jax.nn.sigmoid(z[:, 0:H])
    f = jax.nn.sigmoid(z[:, H:2 * H])
    g = jnp.tanh(z[:, 2 * H:3 * H])
    o = jax.nn.sigmoid(z[:, 3 * H:4 * H])
    c = f * c + i * g
    h = o * jnp.tanh(c)
    return h, c


def _fused(rows, rows_q,
           wxpf, wxpb, whpf, whpb, bpf, bpb,
           wxqf, wxqb, whqf, whqb, bqf, bqb,
           w_att, w_m, b_m2, w_se):
    def body(xf_ref, xb_ref, rq_ref,
             wxpf_ref, wxpb_ref, whpf_ref, whpb_ref, bpf_ref, bpb_ref,
             wxqf_ref, wxqb_ref, whqf_ref, whqb_ref, bqf_ref, bqb_ref,
             wa_ref, wm_ref, bm_ref, wse_ref,
             lo_ref, pr_ref,
             hpf_s, hpb_s, hqf_s, hqb_s, xwf_s, xwb_s,
             hf_s, cf_s, hb_s, cb_s):
        i = pl.program_id(0)

        @pl.when(i == 0)
        def _init():
            z = jnp.zeros((B, H), jnp.float32)
            hf_s[...] = z
            cf_s[...] = z
            hb_s[...] = z
            cb_s[...] = z

        @pl.when(i < NCHP)
        def _scan_p():
            xwf_s[...] = _dot(xf_ref[...], wxpf_ref[...])
            xwb_s[...] = _dot(xb_ref[...], wxpb_ref[...])
            hf, cf, hb, cb = hf_s[...], cf_s[...], hb_s[...], cb_s[...]
            whf = whpf_ref[...]
            whb = whpb_ref[...]
            bf = bpf_ref[...]
            bb = bpb_ref[...]
            for tl in range(CH):
                hf, cf = _lstm_step(xwf_s[tl * B:(tl + 1) * B, :], hf, cf, whf, bf)
                hpf_s[i, :, tl, :] = hf
                tb = CH - 1 - tl
                hb, cb = _lstm_step(xwb_s[tb * B:(tb + 1) * B, :], hb, cb, whb, bb)
                hpb_s[NCHP - 1 - i, :, tb, :] = hb
            hf_s[...] = hf
            cf_s[...] = cf
            hb_s[...] = hb
            cb_s[...] = cb

        @pl.when(i == NCHP)
        def _q_and_attention():
            rq = rq_ref[...].reshape(Q * B, D)
            xwf_s[0:Q * B, :] = _dot(rq, wxqf_ref[...])
            xwb_s[0:Q * B, :] = _dot(rq, wxqb_ref[...])
            z = jnp.zeros((B, H), jnp.float32)
            hf, cf, hb, cb = z, z, z, z
            whf = whqf_ref[...]
            whb = whqb_ref[...]
            bf = bqf_ref[...]
            bb = bqb_ref[...]
            for tl in range(Q):
                hf, cf = _lstm_step(xwf_s[tl * B:(tl + 1) * B, :], hf, cf, whf, bf)
                hqf_s[:, tl, :] = hf
                tb = Q - 1 - tl
                hb, cb = _lstm_step(xwb_s[tb * B:(tb + 1) * B, :], hb, cb, whb, bb)
                hqb_s[:, tb, :] = hb

            wa = wa_ref[...]
            wm = wm_ref[...]
            bm = bm_ref[...]
            wse = wse_ref[...]
            iota = lax.broadcasted_iota(jnp.int32, (2, P), 1)
            GB = 4
            for b0 in range(0, B, GB):
                cats = []
                for b in range(b0, b0 + GB):
                    hp = jnp.concatenate(
                        [hpf_s[:, b, :, :].reshape(P, H),
                         hpb_s[:, b, :, :].reshape(P, H)], axis=-1)
                    hq = jnp.concatenate([hqf_s[b], hqb_s[b]], axis=-1)
                    tmp = _dot_t(hq, wa)
                    scores = _dot_t(hp, tmp)
                    mx = jnp.max(scores, axis=-1, keepdims=True)
                    e = jnp.exp(scores - mx)
                    alpha = e / jnp.sum(e, axis=-1, keepdims=True)
                    ctx = _dot(alpha, hq)
                    cats.append(jnp.concatenate([hp, ctx], axis=-1))
                mg = jnp.tanh(_dot(jnp.concatenate(cats, axis=0), wm) + bm)
                ltg = lax.dot_general(
                    wse, mg, (((0,), (1,)), ((), ())),
                    preferred_element_type=jnp.float32,
                )
                for k in range(GB):
                    lt = ltg[:, k * P:(k + 1) * P]
                    lo_ref[b0 + k, :, :] = lt
                    mx2 = jnp.max(lt, axis=-1, keepdims=True)
                    idx = jnp.min(jnp.where(lt == mx2, iota, P), axis=-1)
                    pr_ref[b0 + k, :, :] = idx.reshape(1, 2)

    c0 = lambda i: (0, 0)
    c03 = lambda i: (0, 0, 0)
    return pl.pallas_call(
        body,
        grid=(NCHP + 1,),
        in_specs=[
            pl.BlockSpec((CH * B, D), lambda i: (jnp.minimum(i, NCHP - 1), 0)),
            pl.BlockSpec((CH * B, D),
                         lambda i: (NCHP - 1 - jnp.minimum(i, NCHP - 1), 0)),
            pl.BlockSpec((Q, B, D), c03),
            pl.BlockSpec((D, 4 * H), c0),
            pl.BlockSpec((D, 4 * H), c0),
            pl.BlockSpec((H, 4 * H), c0),
            pl.BlockSpec((H, 4 * H), c0),
            pl.BlockSpec((1, 4 * H), c0),
            pl.BlockSpec((1, 4 * H), c0),
            pl.BlockSpec((D, 4 * H), c0),
            pl.BlockSpec((D, 4 * H), c0),
            pl.BlockSpec((H, 4 * H), c0),
            pl.BlockSpec((H, 4 * H), c0),
            pl.BlockSpec((1, 4 * H), c0),
            pl.BlockSpec((1, 4 * H), c0),
            pl.BlockSpec((2 * H, 2 * H), c0),
            pl.BlockSpec((4 * H, 2 * H), c0),
            pl.BlockSpec((1, 2 * H), c0),
            pl.BlockSpec((2 * H, 2), c0),
        ],
        out_specs=[
            pl.BlockSpec((B, 2, P), c03),
            pl.BlockSpec((B, 1, 2), c03),
        ],
        out_shape=[
            jax.ShapeDtypeStruct((B, 2, P), jnp.float32),
            jax.ShapeDtypeStruct((B, 1, 2), jnp.int32),
        ],
        scratch_shapes=[
            pltpu.VMEM((NCHP, B, CH, H), jnp.float32),
            pltpu.VMEM((NCHP, B, CH, H), jnp.float32),
            pltpu.VMEM((B, Q, H), jnp.float32),
            pltpu.VMEM((B, Q, H), jnp.float32),
            pltpu.VMEM((CH * B, 4 * H), jnp.float32),
            pltpu.VMEM((CH * B, 4 * H), jnp.float32),
            pltpu.VMEM((B, H), jnp.float32),
            pltpu.VMEM((B, H), jnp.float32),
            pltpu.VMEM((B, H), jnp.float32),
            pltpu.VMEM((B, H), jnp.float32),
        ],
        interpret=_INTERPRET,
    )(rows, rows, rows_q,
      wxpf, wxpb, whpf, whpb, bpf, bpb,
      wxqf, wxqb, whqf, whqb, bqf, bqb,
      w_att, w_m, b_m2, w_se)


def kernel(passage, question, embedding,
           p_Wx_f, p_Wh_f, p_b_f, p_Wx_b, p_Wh_b, p_b_b,
           q_Wx_f, q_Wh_f, q_b_f, q_Wx_b, q_Wh_b, q_b_b,
           W_att, W_m, b_m, w_start, w_end):
    pidx = jnp.transpose(passage).reshape(-1).astype(jnp.int32)
    qidx = jnp.transpose(question[:, ::-1]).reshape(-1).astype(jnp.int32)
    idx = jnp.concatenate([pidx, qidx, jnp.zeros((Q_PAD - NQ_TOK,), jnp.int32)])

    rows = _gather_rows(embedding, idx)
    rows_q = rows[NP_TOK:NP_TOK + NQ_TOK].reshape(Q, B, D)

    logits, preds = _fused(
        rows, rows_q,
        p_Wx_f, p_Wx_b, p_Wh_f, p_Wh_b,
        p_b_f.reshape(1, 4 * H), p_b_b.reshape(1, 4 * H),
        q_Wx_f, q_Wx_b, q_Wh_f, q_Wh_b,
        q_b_f.reshape(1, 4 * H), q_b_b.reshape(1, 4 * H),
        W_att, W_m, b_m.reshape(1, 2 * H), jnp.stack([w_start, w_end], axis=1),
    )
    return logits, preds.reshape(B, 2)
